# baseline (speedup 1.0000x reference)
import jax
import jax.numpy as jnp
from jax import lax
from jax.experimental import pallas as pl
from jax.experimental.pallas import tpu as pltpu
from jax.experimental.pallas import tpu_sc as plsc

_Z = 1.0
_NEWTON_ITERS = 12
_L = 16
_U = 8
_ROWS_PER_TEC = 4


def _sc_body(x_hbm, wp_hbm, wc_hbm, xbuf, wpbuf, wcbuf, sem_wp, sem_wc):
    n = x_hbm.shape[-1]
    nchunks = n // _L
    bank = nchunks
    wid = lax.axis_index("s") * 2 + lax.axis_index("c")
    bankbase = lax.iota(jnp.int32, _L) * bank
    one = jnp.ones((_L,), jnp.int32)
    zero = jnp.zeros((_L,), jnp.int32)

    pending = None
    for r in range(_ROWS_PER_TEC):
        row = wid * _ROWS_PER_TEC + r
        pltpu.sync_copy(x_hbm.at[row], xbuf)

        def fp(g, carry):
            t, ptr = carry
            base = g * (_U * _L)
            for u in range(_U):
                v = xbuf[pl.ds(base + u * _L, _L)]
                t = jnp.maximum(t, v - _Z)
                msk = v > t
                plsc.store_scatter(wcbuf, [ptr], v, mask=msk)
                ptr = ptr + jnp.where(msk, one, zero)
            return (t, ptr)

        t, ptr = lax.fori_loop(
            0,
            nchunks // _U,
            fp,
            (jnp.full((_L,), -jnp.inf, jnp.float32), bankbase),
        )
        tau0 = jnp.full((_L,), jnp.max(t), jnp.float32)
        cnt = ptr - bankbase
        maxcnt = jnp.max(cnt)

        zero_f = jnp.zeros((_L,), jnp.float32)

        def nstep(_, tau):
            def inner(c, acc):
                sv, kv = acc
                v = plsc.load_gather(wcbuf, [bankbase + c])
                act = (c < cnt) & (v > tau)
                sv = sv + jnp.where(act, v, 0.0)
                kv = kv + jnp.where(act, 1.0, 0.0)
                return (sv, kv)

            sv, kv = lax.fori_loop(0, maxcnt, inner, (zero_f, zero_f))
            s = jnp.full((_L,), jnp.sum(sv), jnp.float32)
            k = jnp.full((_L,), jnp.sum(kv), jnp.float32)
            return (s - _Z) / k

        tau = lax.fori_loop(0, _NEWTON_ITERS, nstep, tau0)

        if pending is not None:
            pending[0].wait()
            pending[1].wait()

        def p3(g, _):
            base = g * (_U * _L)
            for u in range(_U):
                sl = pl.ds(base + u * _L, _L)
                v = xbuf[sl]
                wp = jnp.maximum(v - tau, 0.0)
                wpbuf[sl] = wp
                wcbuf[sl] = v - wp
            return 0

        lax.fori_loop(0, nchunks // _U, p3, 0)
        cp_wp = pltpu.async_copy(wpbuf, wp_hbm.at[row], sem_wp)
        cp_wc = pltpu.async_copy(wcbuf, wc_hbm.at[row], sem_wc)
        pending = (cp_wp, cp_wc)

    pending[0].wait()
    pending[1].wait()


def kernel(x):
    b, n = x.shape
    mesh = plsc.VectorSubcoreMesh(core_axis_name="c", subcore_axis_name="s")
    out = jax.ShapeDtypeStruct((b, n), jnp.float32)
    f = pl.kernel(
        _sc_body,
        out_type=(out, out),
        mesh=mesh,
        scratch_types=[
            pltpu.VMEM((n,), jnp.float32),
            pltpu.VMEM((n,), jnp.float32),
            pltpu.VMEM((n,), jnp.float32),
            pltpu.SemaphoreType.DMA,
            pltpu.SemaphoreType.DMA,
        ],
        compiler_params=pltpu.CompilerParams(needs_layout_passes=False),
    )
    return f(x)


# speedup vs baseline: 1.3112x; 1.3112x over previous
import jax
import jax.numpy as jnp
from jax import lax
from jax.experimental import pallas as pl
from jax.experimental.pallas import tpu as pltpu
from jax.experimental.pallas import tpu_sc as plsc

_Z = 1.0
_NEWTON_ITERS = 12
_L = 16
_U = 8
_ROWS_PER_TEC = 4


def _sc_body(x_hbm, wp_hbm, wc_hbm, xbuf, wpbuf, wcbuf, sem_wp, sem_wc):
    n = x_hbm.shape[-1]
    nchunks = n // _L
    wid = lax.axis_index("s") * 2 + lax.axis_index("c")
    lanes = lax.iota(jnp.int32, _L)
    step16 = jnp.full((_L,), _L, jnp.int32)
    zero = jnp.zeros((_L,), jnp.int32)

    pending = None
    for r in range(_ROWS_PER_TEC):
        row = wid * _ROWS_PER_TEC + r
        pltpu.sync_copy(x_hbm.at[row], xbuf)

        if pending is not None:
            pending[1].wait()

        def fp(g, carry):
            t, ptr = carry
            base = g * (_U * _L)
            for u in range(_U):
                v = xbuf[pl.ds(base + u * _L, _L)]
                t = jnp.maximum(t, v - _Z)
                msk = v > t
                plsc.store_scatter(wcbuf, [ptr], v, mask=msk)
                ptr = ptr + jnp.where(msk, step16, zero)
            return (t, ptr)

        t, ptr = lax.fori_loop(
            0,
            nchunks // _U,
            fp,
            (jnp.full((_L,), -jnp.inf, jnp.float32), lanes),
        )
        tau0 = jnp.full((_L,), jnp.max(t), jnp.float32)
        cnt = lax.shift_right_logical(ptr - lanes, 4)
        maxcnt = jnp.max(cnt)

        zero_f = jnp.zeros((_L,), jnp.float32)

        def nstep(_, tau):
            def inner(c, acc):
                sv, kv = acc
                v = wcbuf[pl.ds(c * _L, _L)]
                act = (c < cnt) & (v > tau)
                sv = sv + jnp.where(act, v, 0.0)
                kv = kv + jnp.where(act, 1.0, 0.0)
                return (sv, kv)

            sv, kv = lax.fori_loop(0, maxcnt, inner, (zero_f, zero_f))
            s = jnp.full((_L,), jnp.sum(sv), jnp.float32)
            k = jnp.full((_L,), jnp.sum(kv), jnp.float32)
            return (s - _Z) / k

        tau = lax.fori_loop(0, _NEWTON_ITERS, nstep, tau0)

        if pending is not None:
            pending[0].wait()

        def p3(g, _):
            base = g * (_U * _L)
            for u in range(_U):
                sl = pl.ds(base + u * _L, _L)
                v = xbuf[sl]
                wp = jnp.maximum(v - tau, 0.0)
                wpbuf[sl] = wp
                wcbuf[sl] = v - wp
            return 0

        lax.fori_loop(0, nchunks // _U, p3, 0)
        cp_wp = pltpu.async_copy(wpbuf, wp_hbm.at[row], sem_wp)
        cp_wc = pltpu.async_copy(wcbuf, wc_hbm.at[row], sem_wc)
        pending = (cp_wp, cp_wc)

    pending[0].wait()
    pending[1].wait()


def kernel(x):
    b, n = x.shape
    mesh = plsc.VectorSubcoreMesh(core_axis_name="c", subcore_axis_name="s")
    out = jax.ShapeDtypeStruct((b, n), jnp.float32)
    f = pl.kernel(
        _sc_body,
        out_type=(out, out),
        mesh=mesh,
        scratch_types=[
            pltpu.VMEM((n,), jnp.float32),
            pltpu.VMEM((n,), jnp.float32),
            pltpu.VMEM((n,), jnp.float32),
            pltpu.SemaphoreType.DMA,
            pltpu.SemaphoreType.DMA,
        ],
        compiler_params=pltpu.CompilerParams(needs_layout_passes=False),
    )
    return f(x)
